# baseline (speedup 1.0000x reference)
import functools

import jax
import jax.numpy as jnp
from jax import lax
from jax.experimental import pallas as pl
from jax.experimental.pallas import tpu as pltpu
from jax.experimental.pallas import tpu_sc as plsc

NC = 2
NS = 16
NW = NC * NS

CHUNK = 128
D = 128
SUP = 4


def _sc_edge_agg(n_ent_pad, e_per_w):
    rows_per_tile = n_ent_pad // NS
    n = e_per_w // CHUNK
    assert n % (2 * SUP) == 0 and n >= 4 * SUP
    n_sup = n // SUP
    mesh = plsc.VectorSubcoreMesh(core_axis_name="c", subcore_axis_name="s")

    @functools.partial(
        pl.kernel,
        out_type=(
            jax.ShapeDtypeStruct((NC, n_ent_pad, D), jnp.float32),
            jax.ShapeDtypeStruct((NC, NS, n_ent_pad), jnp.float32),
        ),
        mesh=mesh,
        scratch_types=[
            pltpu.VMEM((16 * D,), jnp.float32),
            pltpu.VMEM((2, 4 * SUP, CHUNK), jnp.int32),
            pltpu.VMEM((2, CHUNK, D), jnp.float32),
            pltpu.VMEM((n_ent_pad,), jnp.float32),
            pltpu.VMEM_SHARED((n_ent_pad, D), jnp.float32),
            pltpu.SemaphoreType.DMA((2,)),
            pltpu.SemaphoreType.DMA((2,)),
            pltpu.SemaphoreType.DMA((2,)),
        ],
        compiler_params=pltpu.CompilerParams(needs_layout_passes=False),
    )
    def k(ent_hbm, pk_hbm, w_hbm, z_hbm,
          acc_hbm, cnt_hbm,
          wv, idxb, rows, counts_v, acc_sh,
          isem, gsem, ssem):
        cid = lax.axis_index("c")
        sid = lax.axis_index("s")
        wid = cid * NS + sid

        RPS = 4 * SUP

        def issue_idx(sup, s):
            off = (wid * n_sup + sup) * RPS
            pltpu.async_copy(pk_hbm.at[pl.ds(off, RPS)], idxb.at[s],
                             isem.at[s])

        def wait_idx(s):
            pltpu.make_async_copy(pk_hbm.at[pl.ds(0, RPS)], idxb.at[s],
                                  isem.at[s]).wait()

        def issue_gather(b, s, r):
            pltpu.async_copy(ent_hbm.at[idxb.at[s, 3 * r + 1]], rows.at[b],
                             gsem.at[b])

        def wait_gather(b, s, r):
            pltpu.make_async_copy(ent_hbm.at[idxb.at[s, 3 * r + 1]],
                                  rows.at[b], gsem.at[b]).wait()

        def issue_scatter(b, s, r):
            pltpu.async_copy(rows.at[b], acc_sh.at[idxb.at[s, 3 * r]],
                             ssem.at[b], add=True)

        def wait_scatter(b, s, r):
            pltpu.make_async_copy(rows.at[b], acc_sh.at[idxb.at[s, 3 * r]],
                                  ssem.at[b]).wait()

        pltpu.sync_copy(z_hbm, acc_sh.at[pl.ds(sid * rows_per_tile, rows_per_tile)])
        pltpu.sync_copy(w_hbm, wv)
        zeros16 = jnp.zeros((16,), jnp.float32)

        def zero_body(i, _):
            counts_v[pl.ds(i * 16, 16)] = zeros16
            return 0

        lax.fori_loop(0, n_ent_pad // 16, zero_body, 0)
        plsc.subcore_barrier()

        def compute(b, s, r):
            rb = rows.at[b]

            def group_body(g, _):
                rel16 = idxb[s, 3 * r + 2, pl.ds(g * 16, 16)]
                wbase = ((rel16 - 1) & 15) * D
                hv16 = idxb[s, 3 * r, pl.ds(g * 16, 16)]
                crun, last = plsc.scan_count(hv16)
                plsc.addupdate_scatter(
                    counts_v, [hv16], crun.astype(jnp.float32), mask=last)
                for l in range(16):
                    e = g * 16 + l
                    wb = wbase[l]
                    for kk in range(D // 16):
                        rb[e, pl.ds(kk * 16, 16)] = (
                            rb[e, pl.ds(kk * 16, 16)]
                            * wv[pl.ds(wb + kk * 16, 16)]
                        )
                return 0

            lax.fori_loop(0, CHUNK // 16, group_body, 0)

        issue_idx(0, 0)
        issue_idx(1, 1)
        wait_idx(0)
        issue_gather(0, 0, 0)

        def octet_body(t, _):
            for jj in range(2 * SUP):
                cc = t * 2 * SUP + jj
                b = jj % 2
                s = jj // SUP
                r = jj % SUP
                pj = jj - 1
                ps, pr = (pj % (2 * SUP)) // SUP, pj % SUP

                wait_gather(b, s, r)
                compute(b, s, r)
                issue_scatter(b, s, r)

                @pl.when(cc >= 1)
                def _():
                    wait_scatter(1 - b, ps, pr)

                if jj == 0:
                    @pl.when((t >= 1) & (SUP * (2 * t + 1) < n))
                    def _():
                        issue_idx(2 * t + 1, 1)
                elif jj == SUP:
                    @pl.when(SUP * (2 * t + 2) < n)
                    def _():
                        issue_idx(2 * t + 2, 0)

                nj = jj + 1
                nss, nr = (nj % (2 * SUP)) // SUP, nj % SUP

                @pl.when(cc + 1 < n)
                def _():
                    if nr == 0:
                        wait_idx(nss)
                    issue_gather(1 - b, nss, nr)
            return 0

        lax.fori_loop(0, n // (2 * SUP), octet_body, 0)
        wait_scatter(1, 1, SUP - 1)
        plsc.subcore_barrier()
        pltpu.sync_copy(
            acc_sh.at[pl.ds(sid * rows_per_tile, rows_per_tile)],
            acc_hbm.at[cid, pl.ds(sid * rows_per_tile, rows_per_tile)],
        )
        pltpu.sync_copy(counts_v, cnt_hbm.at[cid, sid])

    return k


def _combine_kernel(acc_ref, cnt_ref, o_ref):
    s = acc_ref[0] + acc_ref[1]
    cnt = jnp.sum(cnt_ref[...], axis=(0, 1))
    o_ref[...] = s / jnp.maximum(cnt, 1.0)[:, None]


def _mm_kernel(a_ref, b_ref, o_ref):
    o_ref[...] = jnp.dot(a_ref[...], b_ref[...],
                         preferred_element_type=jnp.float32)


def kernel(entity_emb, edge_index, edge_type, interact_mat, weight):
    n_ent, d = entity_emb.shape
    n_users = interact_mat.shape[0]
    n_edges = edge_index.shape[1]
    assert d == D

    n_ent_pad = ((n_ent + 1 + NS * 8 - 1) // (NS * 8)) * (NS * 8)
    rows_per_tile = n_ent_pad // NS

    blk = 2 * SUP * CHUNK
    e_per_w = ((n_edges + NW * blk - 1) // (NW * blk)) * blk
    e_pad = e_per_w * NW
    n = e_per_w // CHUNK
    n_sup = n // SUP

    head = edge_index[0].astype(jnp.int32)
    tail = edge_index[1].astype(jnp.int32)
    rel = edge_type.astype(jnp.int32)
    pad_n = e_pad - n_edges
    head = jnp.concatenate([head, jnp.full((pad_n,), n_ent, jnp.int32)])
    tail = jnp.concatenate([tail, jnp.zeros((pad_n,), jnp.int32)])
    rel = jnp.concatenate([rel, jnp.ones((pad_n,), jnp.int32)])

    h4 = head.reshape(NW, n_sup, SUP, CHUNK)
    t4 = tail.reshape(NW, n_sup, SUP, CHUNK)
    r4 = rel.reshape(NW, n_sup, SUP, CHUNK)
    pk = jnp.stack([h4, t4, r4], axis=3).reshape(NW, n_sup, 3 * SUP, CHUNK)
    pk = jnp.pad(pk, ((0, 0), (0, 0), (0, SUP), (0, 0)))
    pk = pk.reshape(NW * n_sup * 4 * SUP, CHUNK)

    zrows = jnp.zeros((rows_per_tile, D), jnp.float32)
    w_flat = weight.reshape(-1)

    acc, cnt = _sc_edge_agg(n_ent_pad, e_per_w)(
        entity_emb, pk, w_flat, zrows)

    BR = 2048
    entity_agg = pl.pallas_call(
        _combine_kernel,
        grid=(pl.cdiv(n_ent, BR),),
        in_specs=[
            pl.BlockSpec((NC, BR, D), lambda i: (0, i, 0)),
            pl.BlockSpec((NC, NS, BR), lambda i: (0, 0, i)),
        ],
        out_specs=pl.BlockSpec((BR, D), lambda i: (i, 0)),
        out_shape=jax.ShapeDtypeStruct((n_ent, D), jnp.float32),
    )(acc, cnt)

    BM = 256
    user_agg = pl.pallas_call(
        _mm_kernel,
        grid=(n_users // BM,),
        in_specs=[
            pl.BlockSpec((BM, n_ent), lambda i: (i, 0)),
            pl.BlockSpec((n_ent, D), lambda i: (0, 0)),
        ],
        out_specs=pl.BlockSpec((BM, D), lambda i: (i, 0)),
        out_shape=jax.ShapeDtypeStruct((n_users, D), jnp.float32),
        compiler_params=pltpu.CompilerParams(
            dimension_semantics=("arbitrary",)),
    )(interact_mat, entity_emb)

    return (entity_agg, user_agg)


# speedup vs baseline: 1.6454x; 1.6454x over previous
import functools

import jax
import jax.numpy as jnp
from jax import lax
from jax.experimental import pallas as pl
from jax.experimental.pallas import tpu as pltpu
from jax.experimental.pallas import tpu_sc as plsc

NC = 2
NS = 16
NW = NC * NS

CHUNK = 80
D = 128


def _sc_edge_agg(n_ent_pad, e_per_w):
    rows_per_tile = n_ent_pad // NS
    n = e_per_w // CHUNK
    assert n % 3 == 0 and n >= 3
    mesh = plsc.VectorSubcoreMesh(core_axis_name="c", subcore_axis_name="s")

    @functools.partial(
        pl.kernel,
        out_type=(
            jax.ShapeDtypeStruct((NC, n_ent_pad, D), jnp.float32),
            jax.ShapeDtypeStruct((NC, NS, n_ent_pad), jnp.float32),
        ),
        mesh=mesh,
        scratch_types=[
            pltpu.VMEM((16 * D,), jnp.float32),
            pltpu.VMEM((3, CHUNK), jnp.int32),
            pltpu.VMEM((3, CHUNK), jnp.int32),
            pltpu.VMEM((3, CHUNK), jnp.int32),
            pltpu.VMEM((3, CHUNK, D), jnp.float32),
            pltpu.VMEM((n_ent_pad,), jnp.float32),
            pltpu.VMEM_SHARED((n_ent_pad, D), jnp.float32),
            pltpu.SemaphoreType.DMA((3,)),
            pltpu.SemaphoreType.DMA((3,)),
            pltpu.SemaphoreType.DMA((3,)),
        ],
        compiler_params=pltpu.CompilerParams(needs_layout_passes=False),
    )
    def k(ent_hbm, head_hbm, tail_hbm, rel_hbm, w_hbm, z_hbm,
          acc_hbm, cnt_hbm,
          wv, tail_v, head_v, rel_v, rows, counts_v, acc_sh,
          isem, gsem, ssem):
        cid = lax.axis_index("c")
        sid = lax.axis_index("s")
        wid = cid * NS + sid

        def issue_idx(cc, b):
            off = wid * e_per_w + cc * CHUNK
            pltpu.async_copy(head_hbm.at[pl.ds(off, CHUNK)], head_v.at[b],
                             isem.at[b])
            pltpu.async_copy(tail_hbm.at[pl.ds(off, CHUNK)], tail_v.at[b],
                             isem.at[b])
            pltpu.async_copy(rel_hbm.at[pl.ds(off, CHUNK)], rel_v.at[b],
                             isem.at[b])

        def wait_idx(b):
            pltpu.make_async_copy(head_hbm.at[pl.ds(0, CHUNK)], head_v.at[b],
                                  isem.at[b]).wait()
            pltpu.make_async_copy(tail_hbm.at[pl.ds(0, CHUNK)], tail_v.at[b],
                                  isem.at[b]).wait()
            pltpu.make_async_copy(rel_hbm.at[pl.ds(0, CHUNK)], rel_v.at[b],
                                  isem.at[b]).wait()

        def issue_gather(b):
            pltpu.async_copy(ent_hbm.at[tail_v.at[b]], rows.at[b], gsem.at[b])

        def wait_gather(b):
            pltpu.make_async_copy(ent_hbm.at[tail_v.at[b]], rows.at[b],
                                  gsem.at[b]).wait()

        def issue_scatter(b):
            pltpu.async_copy(rows.at[b], acc_sh.at[head_v.at[b]], ssem.at[b],
                             add=True)

        def wait_scatter(b):
            pltpu.make_async_copy(rows.at[b], acc_sh.at[head_v.at[b]],
                                  ssem.at[b]).wait()

        pltpu.sync_copy(z_hbm, acc_sh.at[pl.ds(sid * rows_per_tile, rows_per_tile)])
        pltpu.sync_copy(w_hbm, wv)
        zeros16 = jnp.zeros((16,), jnp.float32)

        def zero_body(i, _):
            counts_v[pl.ds(i * 16, 16)] = zeros16
            return 0

        lax.fori_loop(0, n_ent_pad // 16, zero_body, 0)
        plsc.subcore_barrier()

        def compute(b):
            rb = rows.at[b]

            def group_body(g, _):
                rel16 = rel_v[b, pl.ds(g * 16, 16)]
                wbase = ((rel16 - 1) & 15) * D
                hv16 = head_v[b, pl.ds(g * 16, 16)]
                crun, last = plsc.scan_count(hv16)
                plsc.addupdate_scatter(
                    counts_v, [hv16], crun.astype(jnp.float32), mask=last)
                for l in range(16):
                    e = g * 16 + l
                    wb = wbase[l]
                    for kk in range(D // 16):
                        rb[e, pl.ds(kk * 16, 16)] = (
                            rb[e, pl.ds(kk * 16, 16)]
                            * wv[pl.ds(wb + kk * 16, 16)]
                        )
                return 0

            lax.fori_loop(0, CHUNK // 16, group_body, 0)

        issue_idx(0, 0)
        issue_idx(1, 1)
        wait_idx(0)
        issue_gather(0)

        def tri_body(t, _):
            for j in range(3):
                cc = t * 3 + j
                b, b1, b2 = j, (j + 1) % 3, (j + 2) % 3

                @pl.when(cc + 1 < n)
                def _():
                    wait_idx(b1)
                    issue_gather(b1)

                wait_gather(b)
                compute(b)
                issue_scatter(b)

                @pl.when((cc >= 1) & (cc + 2 < n))
                def _():
                    wait_scatter(b2)

                @pl.when(cc + 2 < n)
                def _():
                    issue_idx(cc + 2, b2)
            return 0

        lax.fori_loop(0, n // 3, tri_body, 0)
        for b in range(3):
            wait_scatter(b)
        plsc.subcore_barrier()
        pltpu.sync_copy(
            acc_sh.at[pl.ds(sid * rows_per_tile, rows_per_tile)],
            acc_hbm.at[cid, pl.ds(sid * rows_per_tile, rows_per_tile)],
        )
        pltpu.sync_copy(counts_v, cnt_hbm.at[cid, sid])

    return k


def _combine_kernel(acc_ref, cnt_ref, o_ref):
    s = acc_ref[0] + acc_ref[1]
    cnt = jnp.sum(cnt_ref[...], axis=(0, 1))
    o_ref[...] = s / jnp.maximum(cnt, 1.0)[:, None]


def _mm_kernel(a_ref, b_ref, o_ref):
    o_ref[...] = jnp.dot(a_ref[...], b_ref[...],
                         preferred_element_type=jnp.float32)


def kernel(entity_emb, edge_index, edge_type, interact_mat, weight):
    n_ent, d = entity_emb.shape
    n_users = interact_mat.shape[0]
    n_edges = edge_index.shape[1]
    assert d == D

    n_ent_pad = ((n_ent + 1 + NS * 8 - 1) // (NS * 8)) * (NS * 8)
    rows_per_tile = n_ent_pad // NS

    e_per_w = ((n_edges + NW * 3 * CHUNK - 1) // (NW * 3 * CHUNK)) * 3 * CHUNK
    e_pad = e_per_w * NW

    head = edge_index[0].astype(jnp.int32)
    tail = edge_index[1].astype(jnp.int32)
    rel = edge_type.astype(jnp.int32)
    pad_n = e_pad - n_edges
    head = jnp.concatenate([head, jnp.full((pad_n,), n_ent, jnp.int32)])
    tail = jnp.concatenate([tail, jnp.zeros((pad_n,), jnp.int32)])
    rel = jnp.concatenate([rel, jnp.ones((pad_n,), jnp.int32)])

    zrows = jnp.zeros((rows_per_tile, D), jnp.float32)
    w_flat = weight.reshape(-1)

    acc, cnt = _sc_edge_agg(n_ent_pad, e_per_w)(
        entity_emb, head, tail, rel, w_flat, zrows)

    BR = 2048
    entity_agg = pl.pallas_call(
        _combine_kernel,
        grid=(pl.cdiv(n_ent, BR),),
        in_specs=[
            pl.BlockSpec((NC, BR, D), lambda i: (0, i, 0)),
            pl.BlockSpec((NC, NS, BR), lambda i: (0, 0, i)),
        ],
        out_specs=pl.BlockSpec((BR, D), lambda i: (i, 0)),
        out_shape=jax.ShapeDtypeStruct((n_ent, D), jnp.float32),
    )(acc, cnt)

    BM = 256
    user_agg = pl.pallas_call(
        _mm_kernel,
        grid=(n_users // BM,),
        in_specs=[
            pl.BlockSpec((BM, n_ent), lambda i: (i, 0)),
            pl.BlockSpec((n_ent, D), lambda i: (0, 0)),
        ],
        out_specs=pl.BlockSpec((BM, D), lambda i: (i, 0)),
        out_shape=jax.ShapeDtypeStruct((n_users, D), jnp.float32),
        compiler_params=pltpu.CompilerParams(
            dimension_semantics=("arbitrary",)),
    )(interact_mat, entity_emb)

    return (entity_agg, user_agg)
